# Initial kernel scaffold; baseline (speedup 1.0000x reference)
import jax
import jax.numpy as jnp
from jax.experimental import pallas as pl


def kernel(query, m_items):
    raise NotImplementedError("write your pallas kernel here")


# speedup vs baseline: 77.7777x; 77.7777x over previous
import functools

import jax
import jax.numpy as jnp
from jax import lax
from jax.experimental import pallas as pl
from jax.experimental.pallas import tpu as pltpu
from jax.experimental.pallas import tpu_sc as plsc

B = 1024
D = 64
M = 100000
SEG = 16
LANES = 128
CHUNK = SEG * LANES
NCHUNK = -(-M // CHUNK)
MP = NCHUNK * CHUNK
NSEG = M // SEG
VALID_LAST = NSEG - (NCHUNK - 1) * LANES
K = 10
NCAND = K * SEG
RB = 128
NEG = -1e30


def _stats_topseg_body(q_ref, m_ref, cv_ref, ci_ref, rmax_ref, rsum_ref):
    c = pl.program_id(0)
    q = q_ref[...].astype(jnp.bfloat16)
    m = m_ref[...].astype(jnp.bfloat16)
    s = lax.dot_general(q, m, (((1,), (1,)), ((), ())),
                        preferred_element_type=jnp.float32)
    nvalid = jnp.where(c == NCHUNK - 1, VALID_LAST, LANES)
    j2048 = lax.broadcasted_iota(jnp.int32, (1, CHUNK), 1) % LANES
    s = jnp.where(j2048 < nvalid, s, NEG)
    segmax = s[:, 0:LANES]
    for t in range(1, SEG):
        segmax = jnp.maximum(segmax, s[:, t * LANES:(t + 1) * LANES])
    chunkmax = jnp.max(segmax, axis=1, keepdims=True)

    @pl.when(c == 0)
    def _():
        rmax_ref[...] = jnp.full((B, 1), NEG, jnp.float32)
        rsum_ref[...] = jnp.zeros((B, 1), jnp.float32)

    prev_max = rmax_ref[...]
    prev_sum = rsum_ref[...]
    new_max = jnp.maximum(prev_max, chunkmax)
    p = jnp.exp(s - new_max)
    psum = jnp.sum(p, axis=1, keepdims=True)
    rsum_ref[...] = prev_sum * jnp.exp(prev_max - new_max) + psum
    rmax_ref[...] = new_max

    lane = lax.broadcasted_iota(jnp.int32, (B, LANES), 1)
    vals = segmax
    for k in range(K):
        mx = jnp.max(vals, axis=1, keepdims=True)
        hit = vals == mx
        sel = jnp.min(jnp.where(hit, lane, LANES), axis=1, keepdims=True)
        cv_ref[0, :, k:k + 1] = mx
        ci_ref[0, :, k:k + 1] = sel + c * LANES
        vals = jnp.where(lane == sel, NEG, vals)
    cv_ref[0, :, K:SEG] = jnp.full((B, SEG - K), NEG, jnp.float32)
    ci_ref[0, :, K:SEG] = jnp.zeros((B, SEG - K), jnp.int32)


def _stats_topseg(query, m_perm, interpret=False):
    return pl.pallas_call(
        _stats_topseg_body,
        grid=(NCHUNK,),
        in_specs=[
            pl.BlockSpec((B, D), lambda c: (0, 0)),
            pl.BlockSpec((CHUNK, D), lambda c: (c, 0)),
        ],
        out_specs=[
            pl.BlockSpec((1, B, SEG), lambda c: (c, 0, 0)),
            pl.BlockSpec((1, B, SEG), lambda c: (c, 0, 0)),
            pl.BlockSpec((B, 1), lambda c: (0, 0)),
            pl.BlockSpec((B, 1), lambda c: (0, 0)),
        ],
        out_shape=[
            jax.ShapeDtypeStruct((NCHUNK, B, SEG), jnp.float32),
            jax.ShapeDtypeStruct((NCHUNK, B, SEG), jnp.int32),
            jax.ShapeDtypeStruct((B, 1), jnp.float32),
            jax.ShapeDtypeStruct((B, 1), jnp.float32),
        ],
        interpret=interpret,
    )(query, m_perm)


def _select_body(cv_ref, ci_ref, sel_ref):
    vals = cv_ref[...]
    ids = ci_ref[...]
    for k in range(K):
        mx = jnp.max(vals, axis=1, keepdims=True)
        hit = vals == mx
        sel = jnp.min(jnp.where(hit, ids, 2 ** 30), axis=1, keepdims=True)
        sel_ref[:, k:k + 1] = sel
        vals = jnp.where(ids == sel, NEG, vals)
    sel_ref[:, K:SEG] = jnp.zeros((B, SEG - K), jnp.int32)


def _select(cvals, cids, interpret=False):
    return pl.pallas_call(
        _select_body,
        out_shape=jax.ShapeDtypeStruct((B, SEG), jnp.int32),
        interpret=interpret,
    )(cvals, cids)


def _sc_gather(table, idx_flat):
    info = plsc.get_sparse_core_info()
    nw = info.num_cores * info.num_subcores
    per_w = (B * K) // nw
    gchunk = 64
    mesh = plsc.VectorSubcoreMesh(core_axis_name="c", subcore_axis_name="s")

    @functools.partial(
        pl.kernel,
        mesh=mesh,
        out_type=jax.ShapeDtypeStruct((B * K, SEG * D), jnp.float32),
        scratch_types=[
            pltpu.VMEM((per_w,), jnp.int32),
            pltpu.VMEM((gchunk, SEG * D), jnp.float32),
            pltpu.SemaphoreType.DMA,
        ],
    )
    def _k(table_hbm, idx_hbm, out_hbm, idx_v, buf, sem):
        wid = lax.axis_index("s") * info.num_cores + lax.axis_index("c")
        base = wid * per_w
        pltpu.sync_copy(idx_hbm.at[pl.ds(base, per_w)], idx_v)
        for g in range(per_w // gchunk):
            pltpu.async_copy(
                table_hbm.at[idx_v.at[pl.ds(g * gchunk, gchunk)]], buf, sem
            ).wait()
            pltpu.sync_copy(buf, out_hbm.at[pl.ds(base + g * gchunk, gchunk)])

    return _k(table, idx_flat)


def _combine_body(q_ref, cand_ref, cols_ref, rmax_ref, rsum_ref, o1_ref, o2_ref):
    q = q_ref[...]
    cand = cand_ref[...]
    cols = cols_ref[...]
    rmax = rmax_ref[...]
    rsum = rsum_ref[...]
    qf = q.astype(jnp.bfloat16).astype(jnp.float32)
    cf = cand.astype(jnp.bfloat16).astype(jnp.float32)
    s = jnp.sum(qf[:, None, :] * cf, axis=2)
    p = jnp.exp(s - rmax) / rsum
    vals = p
    sel_p, sel_c = [], []
    for _ in range(K):
        mx = jnp.max(vals, axis=1, keepdims=True)
        hit = vals == mx
        sc = jnp.min(jnp.where(hit, cols, 2 ** 30), axis=1, keepdims=True)
        sel_p.append(mx)
        sel_c.append(sc)
        vals = jnp.where(cols == sc, NEG, vals)

    def softmax5(ps):
        m = ps[0]
        for x in ps[1:]:
            m = jnp.maximum(m, x)
        es = [jnp.exp(x - m) for x in ps]
        tot = es[0] + es[1] + es[2] + es[3] + es[4]
        return [e / tot for e in es]

    wt = softmax5(sel_p[:5])
    wb = softmax5(sel_p[5:])
    wn = jnp.zeros_like(p)
    wa = jnp.zeros_like(p)
    for k in range(5):
        wn = wn + jnp.where(cols == sel_c[k], wt[k], 0.0)
        wa = wa + jnp.where(cols == sel_c[5 + k], wb[k], 0.0)
    wnf = wn.astype(jnp.bfloat16).astype(jnp.float32)
    waf = wa.astype(jnp.bfloat16).astype(jnp.float32)
    top = jnp.sum(wnf[:, :, None] * cf, axis=1)
    bot = jnp.sum(waf[:, :, None] * cf, axis=1)
    o1_ref[...] = 0.5 * q + 0.5 * top
    o2_ref[...] = 0.01 * q + 0.99 * bot


def _combine(query, cand, cols, rmax, rsum, interpret=False):
    return pl.pallas_call(
        _combine_body,
        grid=(B // RB,),
        in_specs=[
            pl.BlockSpec((RB, D), lambda i: (i, 0)),
            pl.BlockSpec((RB, NCAND, D), lambda i: (i, 0, 0)),
            pl.BlockSpec((RB, NCAND), lambda i: (i, 0)),
            pl.BlockSpec((RB, 1), lambda i: (i, 0)),
            pl.BlockSpec((RB, 1), lambda i: (i, 0)),
        ],
        out_specs=[
            pl.BlockSpec((RB, D), lambda i: (i, 0)),
            pl.BlockSpec((RB, D), lambda i: (i, 0)),
        ],
        out_shape=[
            jax.ShapeDtypeStruct((B, D), jnp.float32),
            jax.ShapeDtypeStruct((B, D), jnp.float32),
        ],
        interpret=interpret,
    )(query, cand, cols, rmax, rsum)


def kernel(query, m_items):
    mp = jnp.pad(m_items, ((0, MP - M), (0, 0)))
    m_perm = mp.reshape(NCHUNK, LANES, SEG, D).transpose(0, 2, 1, 3)
    m_perm = m_perm.reshape(MP, D)

    cvals, cids, rmax, rsum = _stats_topseg(query, m_perm)
    cvals = cvals.transpose(1, 0, 2).reshape(B, NCHUNK * SEG)
    cids = cids.transpose(1, 0, 2).reshape(B, NCHUNK * SEG)
    sel = _select(cvals, cids)

    seg_flat = jnp.clip(sel[:, :K], 0, NSEG - 1).reshape(B * K)
    table = m_items.reshape(NSEG, SEG * D)
    gathered = _sc_gather(table, seg_flat)
    cand = gathered.reshape(B, NCAND, D)

    cols = (seg_flat.reshape(B, K, 1) * SEG
            + jnp.arange(SEG, dtype=jnp.int32).reshape(1, 1, SEG))
    cols = cols.reshape(B, NCAND)
    return _combine(query, cand, cols, rmax, rsum)


# speedup vs baseline: 98.5650x; 1.2673x over previous
import functools

import jax
import jax.numpy as jnp
from jax import lax
from jax.experimental import pallas as pl
from jax.experimental.pallas import tpu as pltpu
from jax.experimental.pallas import tpu_sc as plsc

B = 1024
D = 64
M = 100000
SEG = 16
LANES = 128
CHUNK = SEG * LANES
NCHUNK = -(-M // CHUNK)
NSEGS = NCHUNK * LANES
K = 10
NCAND = K * SEG
RB = 128
NEG = -1e30
COL_SENTINEL = 1.0e7


def _stats_topseg_body(q_ref, m_ref, cv_ref, ci_ref, rmax_ref, rsum_ref):
    c = pl.program_id(0)
    q = q_ref[...].astype(jnp.bfloat16)
    m = m_ref[...].astype(jnp.bfloat16)
    s = lax.dot_general(q, m, (((1,), (1,)), ((), ())),
                        preferred_element_type=jnp.float32)
    ulimit = jnp.where(c == NCHUNK - 1, M - (NCHUNK - 1) * CHUNK, CHUNK)
    u = lax.broadcasted_iota(jnp.int32, (1, CHUNK), 1)
    s = jnp.where(u < ulimit, s, NEG)
    segmax = s[:, 0:LANES]
    for t in range(1, SEG):
        segmax = jnp.maximum(segmax, s[:, t * LANES:(t + 1) * LANES])

    lanef = lax.broadcasted_iota(jnp.int32, (B, LANES), 1).astype(jnp.float32)
    cbase = (c * LANES).astype(jnp.float32)
    vals = segmax
    chunkmax = None
    for k in range(K):
        mx = jnp.max(vals, axis=1, keepdims=True)
        if k == 0:
            chunkmax = mx
        hit = vals == mx
        sel = jnp.min(jnp.where(hit, lanef, 2.0 * LANES), axis=1, keepdims=True)
        cv_ref[0, :, k:k + 1] = mx
        ci_ref[0, :, k:k + 1] = sel + cbase
        vals = jnp.where(lanef == sel, NEG, vals)
    cv_ref[0, :, K:SEG] = jnp.full((B, SEG - K), NEG, jnp.float32)
    ci_ref[0, :, K:SEG] = jnp.zeros((B, SEG - K), jnp.float32)

    @pl.when(c == 0)
    def _():
        rmax_ref[...] = jnp.full((B, 1), NEG, jnp.float32)
        rsum_ref[...] = jnp.zeros((B, 1), jnp.float32)

    prev_max = rmax_ref[...]
    prev_sum = rsum_ref[...]
    new_max = jnp.maximum(prev_max, chunkmax)
    psum = jnp.sum(jnp.exp(s - new_max), axis=1, keepdims=True)
    rsum_ref[...] = prev_sum * jnp.exp(prev_max - new_max) + psum
    rmax_ref[...] = new_max


def _stats_topseg(query, m_items, interpret=False):
    return pl.pallas_call(
        _stats_topseg_body,
        grid=(NCHUNK,),
        in_specs=[
            pl.BlockSpec((B, D), lambda c: (0, 0)),
            pl.BlockSpec((CHUNK, D), lambda c: (c, 0)),
        ],
        out_specs=[
            pl.BlockSpec((1, B, SEG), lambda c: (c, 0, 0)),
            pl.BlockSpec((1, B, SEG), lambda c: (c, 0, 0)),
            pl.BlockSpec((B, 1), lambda c: (0, 0)),
            pl.BlockSpec((B, 1), lambda c: (0, 0)),
        ],
        out_shape=[
            jax.ShapeDtypeStruct((NCHUNK, B, SEG), jnp.float32),
            jax.ShapeDtypeStruct((NCHUNK, B, SEG), jnp.float32),
            jax.ShapeDtypeStruct((B, 1), jnp.float32),
            jax.ShapeDtypeStruct((B, 1), jnp.float32),
        ],
        interpret=interpret,
    )(query, m_items)


def _select_body(cv_ref, ci_ref, sel_ref):
    vals = cv_ref[...]
    ids = ci_ref[...]
    for k in range(K):
        mx = jnp.max(vals, axis=1, keepdims=True)
        hit = vals == mx
        sel = jnp.min(jnp.where(hit, ids, float(2 * NSEGS)), axis=1,
                      keepdims=True)
        sel_ref[:, k:k + 1] = sel.astype(jnp.int32)
        vals = jnp.where(ids == sel, NEG, vals)
    sel_ref[:, K:SEG] = jnp.zeros((B, SEG - K), jnp.int32)


def _select(cvals, cids, interpret=False):
    return pl.pallas_call(
        _select_body,
        out_shape=jax.ShapeDtypeStruct((B, SEG), jnp.int32),
        interpret=interpret,
    )(cvals, cids)


def _sc_gather(table, idx_flat):
    info = plsc.get_sparse_core_info()
    nw = info.num_cores * info.num_subcores
    n = NCAND * B
    per_w = n // nw
    gchunk = 128
    mesh = plsc.VectorSubcoreMesh(core_axis_name="c", subcore_axis_name="s")

    @functools.partial(
        pl.kernel,
        mesh=mesh,
        compiler_params=pltpu.CompilerParams(use_tc_tiling_on_sc=False),
        out_type=jax.ShapeDtypeStruct((n, D), jnp.float32),
        scratch_types=[
            pltpu.VMEM((per_w,), jnp.int32),
            pltpu.VMEM((gchunk, D), jnp.float32),
            pltpu.VMEM((gchunk, D), jnp.float32),
            pltpu.SemaphoreType.DMA,
            pltpu.SemaphoreType.DMA,
        ],
    )
    def _k(table_hbm, idx_hbm, out_hbm, idx_v, buf0, buf1, sem0, sem1):
        wid = lax.axis_index("s") * info.num_cores + lax.axis_index("c")
        base = wid * per_w
        pltpu.sync_copy(idx_hbm.at[pl.ds(base, per_w)], idx_v)
        bufs = (buf0, buf1)
        sems = (sem0, sem1)
        nch = per_w // gchunk
        dmas = []
        for g in range(nch):
            dmas.append(pltpu.async_copy(
                table_hbm.at[idx_v.at[pl.ds(g * gchunk, gchunk)]],
                bufs[g % 2], sems[g % 2]))
            if g >= 1:
                dmas[g - 1].wait()
                pltpu.sync_copy(bufs[(g - 1) % 2],
                                out_hbm.at[pl.ds(base + (g - 1) * gchunk,
                                                 gchunk)])
        dmas[nch - 1].wait()
        pltpu.sync_copy(bufs[(nch - 1) % 2],
                        out_hbm.at[pl.ds(base + (nch - 1) * gchunk, gchunk)])

    return _k(table, idx_flat)


def _combine_body(q_ref, cand_ref, cols_ref, rmax_ref, rsum_ref,
                  o1_ref, o2_ref):
    q = q_ref[...]
    cand = cand_ref[...]
    cols = cols_ref[...]
    rmax = rmax_ref[...]
    rsum = rsum_ref[...]
    qf = q.astype(jnp.bfloat16).astype(jnp.float32)
    cf = cand.astype(jnp.bfloat16).astype(jnp.float32)
    s = jnp.sum(qf[None, :, :] * cf, axis=2)
    p = jnp.exp(s - rmax) / rsum
    vals = jnp.where(cols < COL_SENTINEL, p, NEG)
    sel_p, sel_c = [], []
    for _ in range(K):
        mx = jnp.max(vals, axis=0, keepdims=True)
        hit = vals == mx
        sc = jnp.min(jnp.where(hit, cols, 2.0 * COL_SENTINEL), axis=0,
                     keepdims=True)
        sel_p.append(mx)
        sel_c.append(sc)
        vals = jnp.where(cols == sc, NEG, vals)

    def softmax5(ps):
        m = ps[0]
        for x in ps[1:]:
            m = jnp.maximum(m, x)
        es = [jnp.exp(x - m) for x in ps]
        tot = es[0] + es[1] + es[2] + es[3] + es[4]
        return [e / tot for e in es]

    wt = softmax5(sel_p[:5])
    wb = softmax5(sel_p[5:])
    wn = jnp.zeros_like(p)
    wa = jnp.zeros_like(p)
    for k in range(5):
        wn = wn + jnp.where(cols == sel_c[k], wt[k], 0.0)
        wa = wa + jnp.where(cols == sel_c[5 + k], wb[k], 0.0)
    wnf = wn.astype(jnp.bfloat16).astype(jnp.float32)
    waf = wa.astype(jnp.bfloat16).astype(jnp.float32)
    top = jnp.sum(wnf[:, :, None] * cf, axis=0)
    bot = jnp.sum(waf[:, :, None] * cf, axis=0)
    o1_ref[...] = 0.5 * q + 0.5 * top
    o2_ref[...] = 0.01 * q + 0.99 * bot


def _combine(query, cand, cols, rmaxt, rsumt, interpret=False):
    return pl.pallas_call(
        _combine_body,
        grid=(B // RB,),
        in_specs=[
            pl.BlockSpec((RB, D), lambda i: (i, 0)),
            pl.BlockSpec((NCAND, RB, D), lambda i: (0, i, 0)),
            pl.BlockSpec((NCAND, RB), lambda i: (0, i)),
            pl.BlockSpec((1, RB), lambda i: (0, i)),
            pl.BlockSpec((1, RB), lambda i: (0, i)),
        ],
        out_specs=[
            pl.BlockSpec((RB, D), lambda i: (i, 0)),
            pl.BlockSpec((RB, D), lambda i: (i, 0)),
        ],
        out_shape=[
            jax.ShapeDtypeStruct((B, D), jnp.float32),
            jax.ShapeDtypeStruct((B, D), jnp.float32),
        ],
        interpret=interpret,
    )(query, cand, cols, rmaxt, rsumt)


def _member_cols(sel):
    g = jnp.clip(sel[:, :K], 0, NSEGS - 1).astype(jnp.int32)
    base = (g // LANES) * CHUNK + (g % LANES)
    t = jnp.arange(SEG, dtype=jnp.int32)
    rows = base[:, :, None] + t[None, None, :] * LANES
    rows_t = rows.transpose(1, 2, 0).reshape(NCAND, B)
    valid = rows_t < M
    rows_flat = jnp.where(valid, rows_t, M - 1).reshape(NCAND * B)
    colsT = jnp.where(valid, rows_t.astype(jnp.float32), COL_SENTINEL)
    return rows_flat, colsT


def kernel(query, m_items):
    cvals, cids, rmax, rsum = _stats_topseg(query, m_items)
    cvals = cvals.transpose(1, 0, 2).reshape(B, NCHUNK * SEG)
    cids = cids.transpose(1, 0, 2).reshape(B, NCHUNK * SEG)
    sel = _select(cvals, cids)

    rows_flat, colsT = _member_cols(sel)
    gathered = _sc_gather(m_items, rows_flat)
    cand = gathered.reshape(NCAND, B, D)

    return _combine(query, cand, colsT, rmax.reshape(1, B), rsum.reshape(1, B))


# speedup vs baseline: 113.6797x; 1.1533x over previous
import functools

import jax
import jax.numpy as jnp
from jax import lax
from jax.experimental import pallas as pl
from jax.experimental.pallas import tpu as pltpu
from jax.experimental.pallas import tpu_sc as plsc

B = 1024
D = 64
M = 100000
SEG = 16
LANES = 128
CHUNK = SEG * LANES
NCHUNK = -(-M // CHUNK)
NSEGS = NCHUNK * LANES
K = 10
NCAND = K * SEG
RB = 128
NEG = -1e30
COL_SENTINEL = 1.0e7


def _stats_topseg_body(q_ref, m_ref, sel_ref, stat_ref, cv_s, ci_s):
    c = pl.program_id(0)
    q = q_ref[...].astype(jnp.bfloat16)
    m = m_ref[...].astype(jnp.bfloat16)
    s = lax.dot_general(m, q, (((1,), (1,)), ((), ())),
                        preferred_element_type=jnp.float32)
    ulimit = jnp.where(c == NCHUNK - 1, M - (NCHUNK - 1) * CHUNK, CHUNK)
    u = lax.broadcasted_iota(jnp.int32, (CHUNK, 1), 0)
    s = jnp.where(u < ulimit, s, NEG)
    segmax = s[0:LANES, :]
    for t in range(1, SEG):
        segmax = jnp.maximum(segmax, s[t * LANES:(t + 1) * LANES, :])

    lanef = lax.broadcasted_iota(jnp.int32, (LANES, B), 0).astype(jnp.float32)
    cbase = (c * LANES).astype(jnp.float32)
    vals = segmax
    chunkmax = None
    for k in range(K):
        mx = jnp.max(vals, axis=0, keepdims=True)
        if k == 0:
            chunkmax = mx
        hit = vals == mx
        sel = jnp.min(jnp.where(hit, lanef, 2.0 * LANES), axis=0,
                      keepdims=True)
        cv_s[c, k:k + 1, :] = mx
        ci_s[c, k:k + 1, :] = sel + cbase
        vals = jnp.where(lanef == sel, NEG, vals)

    @pl.when(c == 0)
    def _():
        stat_ref[...] = jnp.full((8, B), NEG, jnp.float32)
        stat_ref[1:2, :] = jnp.zeros((1, B), jnp.float32)

    prev_max = stat_ref[0:1, :]
    prev_sum = stat_ref[1:2, :]
    new_max = jnp.maximum(prev_max, chunkmax)
    psum = jnp.sum(jnp.exp(s - new_max), axis=0, keepdims=True)
    stat_ref[1:2, :] = prev_sum * jnp.exp(prev_max - new_max) + psum
    stat_ref[0:1, :] = new_max

    @pl.when(c == NCHUNK - 1)
    def _():
        v = cv_s[...].reshape(NCHUNK * K, B)
        aids = ci_s[...].reshape(NCHUNK * K, B)
        for k in range(K):
            mx = jnp.max(v, axis=0, keepdims=True)
            hit = v == mx
            g = jnp.min(jnp.where(hit, aids, float(2 * NSEGS)), axis=0,
                        keepdims=True)
            sel_ref[k:k + 1, :] = g.astype(jnp.int32)
            v = jnp.where(aids == g, NEG, v)


def _stats_topseg(query, m_items, interpret=False):
    return pl.pallas_call(
        _stats_topseg_body,
        grid=(NCHUNK,),
        in_specs=[
            pl.BlockSpec((B, D), lambda c: (0, 0)),
            pl.BlockSpec((CHUNK, D), lambda c: (c, 0)),
        ],
        out_specs=[
            pl.BlockSpec((K, B), lambda c: (0, 0)),
            pl.BlockSpec((8, B), lambda c: (0, 0)),
        ],
        out_shape=[
            jax.ShapeDtypeStruct((K, B), jnp.int32),
            jax.ShapeDtypeStruct((8, B), jnp.float32),
        ],
        scratch_shapes=[
            pltpu.VMEM((NCHUNK, K, B), jnp.float32),
            pltpu.VMEM((NCHUNK, K, B), jnp.float32),
        ],
        interpret=interpret,
    )(query, m_items)


def _sc_gather(table, idx_flat):
    info = plsc.get_sparse_core_info()
    nw = info.num_cores * info.num_subcores
    n = NCAND * B
    per_w = n // nw
    gchunk = 128
    mesh = plsc.VectorSubcoreMesh(core_axis_name="c", subcore_axis_name="s")

    @functools.partial(
        pl.kernel,
        mesh=mesh,
        compiler_params=pltpu.CompilerParams(use_tc_tiling_on_sc=False),
        out_type=jax.ShapeDtypeStruct((n, D), jnp.float32),
        scratch_types=[
            pltpu.VMEM((per_w,), jnp.int32),
            pltpu.VMEM((gchunk, D), jnp.float32),
            pltpu.VMEM((gchunk, D), jnp.float32),
            pltpu.SemaphoreType.DMA,
            pltpu.SemaphoreType.DMA,
        ],
    )
    def _k(table_hbm, idx_hbm, out_hbm, idx_v, buf0, buf1, sem0, sem1):
        wid = lax.axis_index("s") * info.num_cores + lax.axis_index("c")
        base = wid * per_w
        pltpu.sync_copy(idx_hbm.at[pl.ds(base, per_w)], idx_v)
        bufs = (buf0, buf1)
        sems = (sem0, sem1)
        nch = per_w // gchunk
        dmas = []
        for g in range(nch):
            dmas.append(pltpu.async_copy(
                table_hbm.at[idx_v.at[pl.ds(g * gchunk, gchunk)]],
                bufs[g % 2], sems[g % 2]))
            if g >= 1:
                dmas[g - 1].wait()
                pltpu.sync_copy(bufs[(g - 1) % 2],
                                out_hbm.at[pl.ds(base + (g - 1) * gchunk,
                                                 gchunk)])
        dmas[nch - 1].wait()
        pltpu.sync_copy(bufs[(nch - 1) % 2],
                        out_hbm.at[pl.ds(base + (nch - 1) * gchunk, gchunk)])

    return _k(table, idx_flat)


def _combine_body(q_ref, cand_ref, cols_ref, rmax_ref, rsum_ref,
                  o1_ref, o2_ref):
    q = q_ref[...]
    cand = cand_ref[...]
    cols = cols_ref[...]
    rmax = rmax_ref[...]
    rsum = rsum_ref[...]
    qf = q.astype(jnp.bfloat16).astype(jnp.float32)
    cf = cand.astype(jnp.bfloat16).astype(jnp.float32)
    s = jnp.sum(qf[None, :, :] * cf, axis=2)
    p = jnp.exp(s - rmax) / rsum
    vals = jnp.where(cols < COL_SENTINEL, p, NEG)
    sel_p, sel_c = [], []
    for _ in range(K):
        mx = jnp.max(vals, axis=0, keepdims=True)
        hit = vals == mx
        sc = jnp.min(jnp.where(hit, cols, 2.0 * COL_SENTINEL), axis=0,
                     keepdims=True)
        sel_p.append(mx)
        sel_c.append(sc)
        vals = jnp.where(cols == sc, NEG, vals)

    def softmax5(ps):
        m = ps[0]
        for x in ps[1:]:
            m = jnp.maximum(m, x)
        es = [jnp.exp(x - m) for x in ps]
        tot = es[0] + es[1] + es[2] + es[3] + es[4]
        return [e / tot for e in es]

    wt = softmax5(sel_p[:5])
    wb = softmax5(sel_p[5:])
    wn = jnp.zeros_like(p)
    wa = jnp.zeros_like(p)
    for k in range(5):
        wn = wn + jnp.where(cols == sel_c[k], wt[k], 0.0)
        wa = wa + jnp.where(cols == sel_c[5 + k], wb[k], 0.0)
    wnf = wn.astype(jnp.bfloat16).astype(jnp.float32)
    waf = wa.astype(jnp.bfloat16).astype(jnp.float32)
    top = jnp.sum(wnf[:, :, None] * cf, axis=0)
    bot = jnp.sum(waf[:, :, None] * cf, axis=0)
    o1_ref[...] = 0.5 * q + 0.5 * top
    o2_ref[...] = 0.01 * q + 0.99 * bot


def _combine(query, cand, cols, rmaxt, rsumt, interpret=False):
    return pl.pallas_call(
        _combine_body,
        grid=(B // RB,),
        in_specs=[
            pl.BlockSpec((RB, D), lambda i: (i, 0)),
            pl.BlockSpec((NCAND, RB, D), lambda i: (0, i, 0)),
            pl.BlockSpec((NCAND, RB), lambda i: (0, i)),
            pl.BlockSpec((1, RB), lambda i: (0, i)),
            pl.BlockSpec((1, RB), lambda i: (0, i)),
        ],
        out_specs=[
            pl.BlockSpec((RB, D), lambda i: (i, 0)),
            pl.BlockSpec((RB, D), lambda i: (i, 0)),
        ],
        out_shape=[
            jax.ShapeDtypeStruct((B, D), jnp.float32),
            jax.ShapeDtypeStruct((B, D), jnp.float32),
        ],
        interpret=interpret,
    )(query, cand, cols, rmaxt, rsumt)


def _member_cols(sel):
    g = jnp.clip(sel, 0, NSEGS - 1).astype(jnp.int32)
    base = (g // LANES) * CHUNK + (g % LANES)
    t = jnp.arange(SEG, dtype=jnp.int32)
    rows = base[:, None, :] + t[None, :, None] * LANES
    rows_t = rows.reshape(NCAND, B)
    valid = rows_t < M
    rows_flat = jnp.where(valid, rows_t, M - 1).reshape(NCAND * B)
    colsT = jnp.where(valid, rows_t.astype(jnp.float32), COL_SENTINEL)
    return rows_flat, colsT


def kernel(query, m_items):
    sel, stat = _stats_topseg(query, m_items)
    rows_flat, colsT = _member_cols(sel)
    gathered = _sc_gather(m_items, rows_flat)
    cand = gathered.reshape(NCAND, B, D)
    return _combine(query, cand, colsT, stat[0:1, :], stat[1:2, :])


# speedup vs baseline: 114.6838x; 1.0088x over previous
import functools

import jax
import jax.numpy as jnp
from jax import lax
from jax.experimental import pallas as pl
from jax.experimental.pallas import tpu as pltpu
from jax.experimental.pallas import tpu_sc as plsc

B = 1024
D = 64
M = 100000
SEG = 16
LANES = 128
CHUNK = SEG * LANES
ACHUNK = 2 * CHUNK
NCHUNK = -(-M // ACHUNK)
SEGS_PER = 2 * LANES
NSEGS = NCHUNK * SEGS_PER
K = 10
NCAND = K * SEG
RB = 128
NEG = -1e30
COL_SENTINEL = 1.0e7


def _stats_topseg_body(q_ref, m_ref, sel_ref, stat_ref, cv_s, ci_s):
    c = pl.program_id(0)
    q = q_ref[...].astype(jnp.bfloat16)
    m = m_ref[...].astype(jnp.bfloat16)
    s = lax.dot_general(m, q, (((1,), (1,)), ((), ())),
                        preferred_element_type=jnp.float32)
    ulimit = jnp.where(c == NCHUNK - 1, M - (NCHUNK - 1) * ACHUNK, ACHUNK)
    u = lax.broadcasted_iota(jnp.int32, (ACHUNK, 1), 0)
    s = jnp.where(u < ulimit, s, NEG)
    halves = []
    for h in range(2):
        hm = s[h * CHUNK:h * CHUNK + LANES, :]
        for t in range(1, SEG):
            hm = jnp.maximum(
                hm, s[h * CHUNK + t * LANES:h * CHUNK + (t + 1) * LANES, :])
        halves.append(hm)
    segmax = jnp.concatenate(halves, axis=0)

    lanef = lax.broadcasted_iota(jnp.int32, (SEGS_PER, B), 0).astype(
        jnp.float32)
    cbase = (c * SEGS_PER).astype(jnp.float32)
    vals = segmax
    chunkmax = None
    for k in range(K):
        mx = jnp.max(vals, axis=0, keepdims=True)
        if k == 0:
            chunkmax = mx
        hit = vals == mx
        sel = jnp.min(jnp.where(hit, lanef, 2.0 * SEGS_PER), axis=0,
                      keepdims=True)
        cv_s[c, k:k + 1, :] = mx
        ci_s[c, k:k + 1, :] = sel + cbase
        vals = jnp.where(lanef == sel, NEG, vals)

    @pl.when(c == 0)
    def _():
        stat_ref[...] = jnp.full((8, B), NEG, jnp.float32)
        stat_ref[1:2, :] = jnp.zeros((1, B), jnp.float32)

    prev_max = stat_ref[0:1, :]
    prev_sum = stat_ref[1:2, :]
    new_max = jnp.maximum(prev_max, chunkmax)
    psum = jnp.sum(jnp.exp(s - new_max), axis=0, keepdims=True)
    stat_ref[1:2, :] = prev_sum * jnp.exp(prev_max - new_max) + psum
    stat_ref[0:1, :] = new_max

    @pl.when(c == NCHUNK - 1)
    def _():
        v = cv_s[...].reshape(NCHUNK * K, B)
        aids = ci_s[...].reshape(NCHUNK * K, B)
        for k in range(K):
            mx = jnp.max(v, axis=0, keepdims=True)
            hit = v == mx
            g = jnp.min(jnp.where(hit, aids, float(2 * NSEGS)), axis=0,
                        keepdims=True)
            sel_ref[k:k + 1, :] = g.astype(jnp.int32)
            v = jnp.where(aids == g, NEG, v)


def _stats_topseg(query, m_items, interpret=False):
    return pl.pallas_call(
        _stats_topseg_body,
        grid=(NCHUNK,),
        in_specs=[
            pl.BlockSpec((B, D), lambda c: (0, 0)),
            pl.BlockSpec((ACHUNK, D), lambda c: (c, 0)),
        ],
        out_specs=[
            pl.BlockSpec((K, B), lambda c: (0, 0)),
            pl.BlockSpec((8, B), lambda c: (0, 0)),
        ],
        out_shape=[
            jax.ShapeDtypeStruct((K, B), jnp.int32),
            jax.ShapeDtypeStruct((8, B), jnp.float32),
        ],
        scratch_shapes=[
            pltpu.VMEM((NCHUNK, K, B), jnp.float32),
            pltpu.VMEM((NCHUNK, K, B), jnp.float32),
        ],
        interpret=interpret,
    )(query, m_items)


def _sc_gather(table, idx_flat):
    info = plsc.get_sparse_core_info()
    nw = info.num_cores * info.num_subcores
    n = NCAND * B
    per_w = n // nw
    gchunk = 128
    mesh = plsc.VectorSubcoreMesh(core_axis_name="c", subcore_axis_name="s")

    @functools.partial(
        pl.kernel,
        mesh=mesh,
        compiler_params=pltpu.CompilerParams(use_tc_tiling_on_sc=False),
        out_type=jax.ShapeDtypeStruct((n, D), jnp.float32),
        scratch_types=[
            pltpu.VMEM((per_w,), jnp.int32),
            pltpu.VMEM((gchunk, D), jnp.float32),
            pltpu.VMEM((gchunk, D), jnp.float32),
            pltpu.SemaphoreType.DMA,
            pltpu.SemaphoreType.DMA,
        ],
    )
    def _k(table_hbm, idx_hbm, out_hbm, idx_v, buf0, buf1, sem0, sem1):
        wid = lax.axis_index("s") * info.num_cores + lax.axis_index("c")
        base = wid * per_w
        pltpu.sync_copy(idx_hbm.at[pl.ds(base, per_w)], idx_v)
        bufs = (buf0, buf1)
        sems = (sem0, sem1)
        nch = per_w // gchunk
        dmas = []
        for g in range(nch):
            dmas.append(pltpu.async_copy(
                table_hbm.at[idx_v.at[pl.ds(g * gchunk, gchunk)]],
                bufs[g % 2], sems[g % 2]))
            if g >= 1:
                dmas[g - 1].wait()
                pltpu.sync_copy(bufs[(g - 1) % 2],
                                out_hbm.at[pl.ds(base + (g - 1) * gchunk,
                                                 gchunk)])
        dmas[nch - 1].wait()
        pltpu.sync_copy(bufs[(nch - 1) % 2],
                        out_hbm.at[pl.ds(base + (nch - 1) * gchunk, gchunk)])

    return _k(table, idx_flat)


def _combine_body(q_ref, cand_ref, cols_ref, rmax_ref, rsum_ref,
                  o1_ref, o2_ref):
    q = q_ref[...]
    cand = cand_ref[...]
    cols = cols_ref[...]
    rmax = rmax_ref[...]
    rsum = rsum_ref[...]
    qf = q.astype(jnp.bfloat16).astype(jnp.float32)
    cf = cand.astype(jnp.bfloat16).astype(jnp.float32)
    s = jnp.sum(qf[None, :, :] * cf, axis=2)
    p = jnp.exp(s - rmax) / rsum
    vals = jnp.where(cols < COL_SENTINEL, p, NEG)
    sel_p, sel_c = [], []
    for _ in range(K):
        mx = jnp.max(vals, axis=0, keepdims=True)
        hit = vals == mx
        sc = jnp.min(jnp.where(hit, cols, 2.0 * COL_SENTINEL), axis=0,
                     keepdims=True)
        sel_p.append(mx)
        sel_c.append(sc)
        vals = jnp.where(cols == sc, NEG, vals)

    def softmax5(ps):
        m = ps[0]
        for x in ps[1:]:
            m = jnp.maximum(m, x)
        es = [jnp.exp(x - m) for x in ps]
        tot = es[0] + es[1] + es[2] + es[3] + es[4]
        return [e / tot for e in es]

    wt = softmax5(sel_p[:5])
    wb = softmax5(sel_p[5:])
    wn = jnp.zeros_like(p)
    wa = jnp.zeros_like(p)
    for k in range(5):
        wn = wn + jnp.where(cols == sel_c[k], wt[k], 0.0)
        wa = wa + jnp.where(cols == sel_c[5 + k], wb[k], 0.0)
    wnf = wn.astype(jnp.bfloat16).astype(jnp.float32)
    waf = wa.astype(jnp.bfloat16).astype(jnp.float32)
    top = jnp.sum(wnf[:, :, None] * cf, axis=0)
    bot = jnp.sum(waf[:, :, None] * cf, axis=0)
    o1_ref[...] = 0.5 * q + 0.5 * top
    o2_ref[...] = 0.01 * q + 0.99 * bot


def _combine(query, cand, cols, rmaxt, rsumt, interpret=False):
    return pl.pallas_call(
        _combine_body,
        grid=(B // RB,),
        in_specs=[
            pl.BlockSpec((RB, D), lambda i: (i, 0)),
            pl.BlockSpec((NCAND, RB, D), lambda i: (0, i, 0)),
            pl.BlockSpec((NCAND, RB), lambda i: (0, i)),
            pl.BlockSpec((1, RB), lambda i: (0, i)),
            pl.BlockSpec((1, RB), lambda i: (0, i)),
        ],
        out_specs=[
            pl.BlockSpec((RB, D), lambda i: (i, 0)),
            pl.BlockSpec((RB, D), lambda i: (i, 0)),
        ],
        out_shape=[
            jax.ShapeDtypeStruct((B, D), jnp.float32),
            jax.ShapeDtypeStruct((B, D), jnp.float32),
        ],
        interpret=interpret,
    )(query, cand, cols, rmaxt, rsumt)


def _member_cols(sel):
    g = jnp.clip(sel, 0, NSEGS - 1).astype(jnp.int32)
    base = (g // LANES) * CHUNK + (g % LANES)
    t = jnp.arange(SEG, dtype=jnp.int32)
    rows = base[:, None, :] + t[None, :, None] * LANES
    rows_t = rows.reshape(NCAND, B)
    valid = rows_t < M
    rows_flat = jnp.where(valid, rows_t, M - 1).reshape(NCAND * B)
    colsT = jnp.where(valid, rows_t.astype(jnp.float32), COL_SENTINEL)
    return rows_flat, colsT


def kernel(query, m_items):
    sel, stat = _stats_topseg(query, m_items)
    rows_flat, colsT = _member_cols(sel)
    gathered = _sc_gather(m_items, rows_flat)
    cand = gathered.reshape(NCAND, B, D)
    return _combine(query, cand, colsT, stat[0:1, :], stat[1:2, :])


# speedup vs baseline: 116.2721x; 1.0138x over previous
import functools

import jax
import jax.numpy as jnp
from jax import lax
from jax.experimental import pallas as pl
from jax.experimental.pallas import tpu as pltpu
from jax.experimental.pallas import tpu_sc as plsc

B = 512
BFULL = 1024
D = 64
M = 100000
SEG = 16
LANES = 128
CHUNK = SEG * LANES
ACHUNK = 2 * CHUNK
NCHUNK = -(-M // ACHUNK)
SEGS_PER = 2 * LANES
NSEGS = NCHUNK * SEGS_PER
K = 10
NCAND = K * SEG
RB = 128
NEG = -1e30
COL_SENTINEL = 1.0e7


def _stats_topseg_body(q_ref, m_ref, sel_ref, stat_ref, cv_s, ci_s):
    c = pl.program_id(0)
    q = q_ref[...].astype(jnp.bfloat16)
    m = m_ref[...].astype(jnp.bfloat16)
    s = lax.dot_general(m, q, (((1,), (1,)), ((), ())),
                        preferred_element_type=jnp.float32)
    ulimit = jnp.where(c == NCHUNK - 1, M - (NCHUNK - 1) * ACHUNK, ACHUNK)
    u = lax.broadcasted_iota(jnp.int32, (ACHUNK, 1), 0)
    s = jnp.where(u < ulimit, s, NEG)
    halves = []
    for h in range(2):
        hm = s[h * CHUNK:h * CHUNK + LANES, :]
        for t in range(1, SEG):
            hm = jnp.maximum(
                hm, s[h * CHUNK + t * LANES:h * CHUNK + (t + 1) * LANES, :])
        halves.append(hm)
    segmax = jnp.concatenate(halves, axis=0)

    lanef = lax.broadcasted_iota(jnp.int32, (SEGS_PER, B), 0).astype(
        jnp.float32)
    cbase = (c * SEGS_PER).astype(jnp.float32)
    vals = segmax
    chunkmax = None
    for k in range(K):
        mx = jnp.max(vals, axis=0, keepdims=True)
        if k == 0:
            chunkmax = mx
        hit = vals == mx
        sel = jnp.min(jnp.where(hit, lanef, 2.0 * SEGS_PER), axis=0,
                      keepdims=True)
        cv_s[c, k:k + 1, :] = mx
        ci_s[c, k:k + 1, :] = sel + cbase
        vals = jnp.where(lanef == sel, NEG, vals)

    @pl.when(c == 0)
    def _():
        stat_ref[...] = jnp.full((8, B), NEG, jnp.float32)
        stat_ref[1:2, :] = jnp.zeros((1, B), jnp.float32)

    prev_max = stat_ref[0:1, :]
    prev_sum = stat_ref[1:2, :]
    new_max = jnp.maximum(prev_max, chunkmax)
    psum = jnp.sum(jnp.exp(s - new_max), axis=0, keepdims=True)
    stat_ref[1:2, :] = prev_sum * jnp.exp(prev_max - new_max) + psum
    stat_ref[0:1, :] = new_max

    @pl.when(c == NCHUNK - 1)
    def _():
        v = cv_s[...].reshape(NCHUNK * K, B)
        aids = ci_s[...].reshape(NCHUNK * K, B)
        for k in range(K):
            mx = jnp.max(v, axis=0, keepdims=True)
            hit = v == mx
            g = jnp.min(jnp.where(hit, aids, float(2 * NSEGS)), axis=0,
                        keepdims=True)
            sel_ref[k:k + 1, :] = g.astype(jnp.int32)
            v = jnp.where(aids == g, NEG, v)


def _stats_topseg(query, m_items, interpret=False):
    return pl.pallas_call(
        _stats_topseg_body,
        grid=(NCHUNK,),
        in_specs=[
            pl.BlockSpec((B, D), lambda c: (0, 0)),
            pl.BlockSpec((ACHUNK, D), lambda c: (c, 0)),
        ],
        out_specs=[
            pl.BlockSpec((K, B), lambda c: (0, 0)),
            pl.BlockSpec((8, B), lambda c: (0, 0)),
        ],
        out_shape=[
            jax.ShapeDtypeStruct((K, B), jnp.int32),
            jax.ShapeDtypeStruct((8, B), jnp.float32),
        ],
        scratch_shapes=[
            pltpu.VMEM((NCHUNK, K, B), jnp.float32),
            pltpu.VMEM((NCHUNK, K, B), jnp.float32),
        ],
        interpret=interpret,
    )(query, m_items)


def _sc_gather(table, idx_flat):
    info = plsc.get_sparse_core_info()
    nw = info.num_cores * info.num_subcores
    n = NCAND * B
    per_w = n // nw
    gchunk = 128
    mesh = plsc.VectorSubcoreMesh(core_axis_name="c", subcore_axis_name="s")

    @functools.partial(
        pl.kernel,
        mesh=mesh,
        compiler_params=pltpu.CompilerParams(use_tc_tiling_on_sc=False),
        out_type=jax.ShapeDtypeStruct((n, D), jnp.float32),
        scratch_types=[
            pltpu.VMEM((per_w,), jnp.int32),
            pltpu.VMEM((gchunk, D), jnp.float32),
            pltpu.VMEM((gchunk, D), jnp.float32),
            pltpu.SemaphoreType.DMA,
            pltpu.SemaphoreType.DMA,
        ],
    )
    def _k(table_hbm, idx_hbm, out_hbm, idx_v, buf0, buf1, sem0, sem1):
        wid = lax.axis_index("s") * info.num_cores + lax.axis_index("c")
        base = wid * per_w
        pltpu.sync_copy(idx_hbm.at[pl.ds(base, per_w)], idx_v)
        bufs = (buf0, buf1)
        sems = (sem0, sem1)
        nch = per_w // gchunk
        dmas = []
        for g in range(nch):
            dmas.append(pltpu.async_copy(
                table_hbm.at[idx_v.at[pl.ds(g * gchunk, gchunk)]],
                bufs[g % 2], sems[g % 2]))
            if g >= 1:
                dmas[g - 1].wait()
                pltpu.sync_copy(bufs[(g - 1) % 2],
                                out_hbm.at[pl.ds(base + (g - 1) * gchunk,
                                                 gchunk)])
        dmas[nch - 1].wait()
        pltpu.sync_copy(bufs[(nch - 1) % 2],
                        out_hbm.at[pl.ds(base + (nch - 1) * gchunk, gchunk)])

    return _k(table, idx_flat)


def _combine_body(q_ref, cand_ref, cols_ref, rmax_ref, rsum_ref,
                  o1_ref, o2_ref):
    q = q_ref[...]
    cand = cand_ref[...]
    cols = cols_ref[...]
    rmax = rmax_ref[...]
    rsum = rsum_ref[...]
    qf = q.astype(jnp.bfloat16).astype(jnp.float32)
    cf = cand.astype(jnp.bfloat16).astype(jnp.float32)
    s = jnp.sum(qf[None, :, :] * cf, axis=2)
    p = jnp.exp(s - rmax) / rsum
    vals = jnp.where(cols < COL_SENTINEL, p, NEG)
    sel_p, sel_c = [], []
    for _ in range(K):
        mx = jnp.max(vals, axis=0, keepdims=True)
        hit = vals == mx
        sc = jnp.min(jnp.where(hit, cols, 2.0 * COL_SENTINEL), axis=0,
                     keepdims=True)
        sel_p.append(mx)
        sel_c.append(sc)
        vals = jnp.where(cols == sc, NEG, vals)

    def softmax5(ps):
        m = ps[0]
        for x in ps[1:]:
            m = jnp.maximum(m, x)
        es = [jnp.exp(x - m) for x in ps]
        tot = es[0] + es[1] + es[2] + es[3] + es[4]
        return [e / tot for e in es]

    wt = softmax5(sel_p[:5])
    wb = softmax5(sel_p[5:])
    wn = jnp.zeros_like(p)
    wa = jnp.zeros_like(p)
    for k in range(5):
        wn = wn + jnp.where(cols == sel_c[k], wt[k], 0.0)
        wa = wa + jnp.where(cols == sel_c[5 + k], wb[k], 0.0)
    wnf = wn.astype(jnp.bfloat16).astype(jnp.float32)
    waf = wa.astype(jnp.bfloat16).astype(jnp.float32)
    top = jnp.sum(wnf[:, :, None] * cf, axis=0)
    bot = jnp.sum(waf[:, :, None] * cf, axis=0)
    o1_ref[...] = 0.5 * q + 0.5 * top
    o2_ref[...] = 0.01 * q + 0.99 * bot


def _combine(query, cand, cols, rmaxt, rsumt, interpret=False):
    return pl.pallas_call(
        _combine_body,
        grid=(B // RB,),
        in_specs=[
            pl.BlockSpec((RB, D), lambda i: (i, 0)),
            pl.BlockSpec((NCAND, RB, D), lambda i: (0, i, 0)),
            pl.BlockSpec((NCAND, RB), lambda i: (0, i)),
            pl.BlockSpec((1, RB), lambda i: (0, i)),
            pl.BlockSpec((1, RB), lambda i: (0, i)),
        ],
        out_specs=[
            pl.BlockSpec((RB, D), lambda i: (i, 0)),
            pl.BlockSpec((RB, D), lambda i: (i, 0)),
        ],
        out_shape=[
            jax.ShapeDtypeStruct((B, D), jnp.float32),
            jax.ShapeDtypeStruct((B, D), jnp.float32),
        ],
        interpret=interpret,
    )(query, cand, cols, rmaxt, rsumt)


def _member_cols(sel):
    g = jnp.clip(sel, 0, NSEGS - 1).astype(jnp.int32)
    base = (g // LANES) * CHUNK + (g % LANES)
    t = jnp.arange(SEG, dtype=jnp.int32)
    rows = base[:, None, :] + t[None, :, None] * LANES
    rows_t = rows.reshape(NCAND, B)
    valid = rows_t < M
    rows_flat = jnp.where(valid, rows_t, M - 1).reshape(NCAND * B)
    colsT = jnp.where(valid, rows_t.astype(jnp.float32), COL_SENTINEL)
    return rows_flat, colsT


def _half(query, m_items):
    sel, stat = _stats_topseg(query, m_items)
    rows_flat, colsT = _member_cols(sel)
    gathered = _sc_gather(m_items, rows_flat)
    cand = gathered.reshape(NCAND, B, D)
    return _combine(query, cand, colsT, stat[0:1, :], stat[1:2, :])


def kernel(query, m_items):
    o1a, o2a = _half(query[:B], m_items)
    o1b, o2b = _half(query[B:], m_items)
    return (jnp.concatenate([o1a, o1b], axis=0),
            jnp.concatenate([o2a, o2b], axis=0))


# speedup vs baseline: 116.4392x; 1.0014x over previous
import functools

import jax
import jax.numpy as jnp
from jax import lax
from jax.experimental import pallas as pl
from jax.experimental.pallas import tpu as pltpu
from jax.experimental.pallas import tpu_sc as plsc

B = 512
BFULL = 1024
D = 64
M = 100000
SEG = 16
LANES = 128
CHUNK = SEG * LANES
ACHUNK = 2 * CHUNK
NCHUNK = -(-M // ACHUNK)
SEGS_PER = 2 * LANES
NSEGS = NCHUNK * SEGS_PER
K = 10
NCAND = K * SEG
RB = 128
NEG = -1e30
COL_SENTINEL = 1.0e7


def _stats_topseg_body(q_ref, m_ref, sel_ref, stat_ref, cv_s, ci_s):
    c = pl.program_id(0)
    q = q_ref[...].astype(jnp.bfloat16)
    m = m_ref[...].astype(jnp.bfloat16)
    s = lax.dot_general(m, q, (((1,), (1,)), ((), ())),
                        preferred_element_type=jnp.float32)
    ulimit = jnp.where(c == NCHUNK - 1, M - (NCHUNK - 1) * ACHUNK, ACHUNK)
    u = lax.broadcasted_iota(jnp.int32, (ACHUNK, 1), 0)
    s = jnp.where(u < ulimit, s, NEG)
    halves = []
    for h in range(2):
        hm = s[h * CHUNK:h * CHUNK + LANES, :]
        for t in range(1, SEG):
            hm = jnp.maximum(
                hm, s[h * CHUNK + t * LANES:h * CHUNK + (t + 1) * LANES, :])
        halves.append(hm)
    segmax = jnp.concatenate(halves, axis=0)

    lanef = lax.broadcasted_iota(jnp.int32, (SEGS_PER, B), 0).astype(
        jnp.float32)
    cbase = (c * SEGS_PER).astype(jnp.float32)
    vals = segmax
    chunkmax = None
    for k in range(K):
        mx = jnp.max(vals, axis=0, keepdims=True)
        if k == 0:
            chunkmax = mx
        hit = vals == mx
        sel = jnp.min(jnp.where(hit, lanef, 2.0 * SEGS_PER), axis=0,
                      keepdims=True)
        cv_s[c, k:k + 1, :] = mx
        ci_s[c, k:k + 1, :] = sel + cbase
        vals = jnp.where(lanef == sel, NEG, vals)

    @pl.when(c == 0)
    def _():
        stat_ref[...] = jnp.full((8, B), NEG, jnp.float32)
        stat_ref[1:2, :] = jnp.zeros((1, B), jnp.float32)

    prev_max = stat_ref[0:1, :]
    prev_sum = stat_ref[1:2, :]
    new_max = jnp.maximum(prev_max, chunkmax)
    psum = jnp.sum(jnp.exp(s - new_max), axis=0, keepdims=True)
    stat_ref[1:2, :] = prev_sum * jnp.exp(prev_max - new_max) + psum
    stat_ref[0:1, :] = new_max

    @pl.when(c == NCHUNK - 1)
    def _():
        v = cv_s[...].reshape(NCHUNK * K, B)
        aids = ci_s[...].reshape(NCHUNK * K, B)
        for k in range(K):
            mx = jnp.max(v, axis=0, keepdims=True)
            hit = v == mx
            g = jnp.min(jnp.where(hit, aids, float(2 * NSEGS)), axis=0,
                        keepdims=True)
            sel_ref[k:k + 1, :] = g.astype(jnp.int32)
            v = jnp.where(aids == g, NEG, v)


def _stats_topseg(query, m_items, interpret=False):
    return pl.pallas_call(
        _stats_topseg_body,
        grid=(NCHUNK,),
        in_specs=[
            pl.BlockSpec((B, D), lambda c: (0, 0)),
            pl.BlockSpec((ACHUNK, D), lambda c: (c, 0)),
        ],
        out_specs=[
            pl.BlockSpec((K, B), lambda c: (0, 0)),
            pl.BlockSpec((8, B), lambda c: (0, 0)),
        ],
        out_shape=[
            jax.ShapeDtypeStruct((K, B), jnp.int32),
            jax.ShapeDtypeStruct((8, B), jnp.float32),
        ],
        scratch_shapes=[
            pltpu.VMEM((NCHUNK, K, B), jnp.float32),
            pltpu.VMEM((NCHUNK, K, B), jnp.float32),
        ],
        interpret=interpret,
    )(query, m_items)


def _sc_gather(table, idx_flat):
    info = plsc.get_sparse_core_info()
    nw = info.num_cores * info.num_subcores
    n = NCAND * B
    per_w = n // nw
    gchunk = 128
    mesh = plsc.VectorSubcoreMesh(core_axis_name="c", subcore_axis_name="s")

    @functools.partial(
        pl.kernel,
        mesh=mesh,
        compiler_params=pltpu.CompilerParams(use_tc_tiling_on_sc=False),
        out_type=jax.ShapeDtypeStruct((n, D), jnp.float32),
        scratch_types=[
            pltpu.VMEM((per_w,), jnp.int32),
            pltpu.VMEM((gchunk, D), jnp.float32),
            pltpu.VMEM((gchunk, D), jnp.float32),
            pltpu.SemaphoreType.DMA,
            pltpu.SemaphoreType.DMA,
            pltpu.SemaphoreType.DMA,
            pltpu.SemaphoreType.DMA,
        ],
    )
    def _k(table_hbm, idx_hbm, out_hbm, idx_v, buf0, buf1, sem0, sem1,
           osem0, osem1):
        wid = lax.axis_index("s") * info.num_cores + lax.axis_index("c")
        base = wid * per_w
        pltpu.sync_copy(idx_hbm.at[pl.ds(base, per_w)], idx_v)
        bufs = (buf0, buf1)
        sems = (sem0, sem1)
        osems = (osem0, osem1)
        nch = per_w // gchunk
        gd, sd = [], []
        for g in range(nch):
            if g >= 2:
                sd[g - 2].wait()
            gd.append(pltpu.async_copy(
                table_hbm.at[idx_v.at[pl.ds(g * gchunk, gchunk)]],
                bufs[g % 2], sems[g % 2]))
            if g >= 1:
                gd[g - 1].wait()
                sd.append(pltpu.async_copy(
                    bufs[(g - 1) % 2],
                    out_hbm.at[pl.ds(base + (g - 1) * gchunk, gchunk)],
                    osems[(g - 1) % 2]))
        gd[nch - 1].wait()
        sd.append(pltpu.async_copy(
            bufs[(nch - 1) % 2],
            out_hbm.at[pl.ds(base + (nch - 1) * gchunk, gchunk)],
            osems[(nch - 1) % 2]))
        sd[nch - 2].wait()
        sd[nch - 1].wait()

    return _k(table, idx_flat)


def _combine_body(q_ref, cand_ref, cols_ref, rmax_ref, rsum_ref,
                  o1_ref, o2_ref):
    q = q_ref[...]
    cand = cand_ref[...]
    cols = cols_ref[...]
    rmax = rmax_ref[...]
    rsum = rsum_ref[...]
    qf = q.astype(jnp.bfloat16).astype(jnp.float32)
    cf = cand.astype(jnp.bfloat16).astype(jnp.float32)
    s = jnp.sum(qf[None, :, :] * cf, axis=2)
    p = jnp.exp(s - rmax) / rsum
    vals = jnp.where(cols < COL_SENTINEL, p, NEG)
    sel_p, sel_c = [], []
    for _ in range(K):
        mx = jnp.max(vals, axis=0, keepdims=True)
        hit = vals == mx
        sc = jnp.min(jnp.where(hit, cols, 2.0 * COL_SENTINEL), axis=0,
                     keepdims=True)
        sel_p.append(mx)
        sel_c.append(sc)
        vals = jnp.where(cols == sc, NEG, vals)

    def softmax5(ps):
        m = ps[0]
        for x in ps[1:]:
            m = jnp.maximum(m, x)
        es = [jnp.exp(x - m) for x in ps]
        tot = es[0] + es[1] + es[2] + es[3] + es[4]
        return [e / tot for e in es]

    wt = softmax5(sel_p[:5])
    wb = softmax5(sel_p[5:])
    wn = jnp.zeros_like(p)
    wa = jnp.zeros_like(p)
    for k in range(5):
        wn = wn + jnp.where(cols == sel_c[k], wt[k], 0.0)
        wa = wa + jnp.where(cols == sel_c[5 + k], wb[k], 0.0)
    wnf = wn.astype(jnp.bfloat16).astype(jnp.float32)
    waf = wa.astype(jnp.bfloat16).astype(jnp.float32)
    top = jnp.sum(wnf[:, :, None] * cf, axis=0)
    bot = jnp.sum(waf[:, :, None] * cf, axis=0)
    o1_ref[...] = 0.5 * q + 0.5 * top
    o2_ref[...] = 0.01 * q + 0.99 * bot


def _combine(query, cand, cols, rmaxt, rsumt, interpret=False):
    return pl.pallas_call(
        _combine_body,
        grid=(B // RB,),
        in_specs=[
            pl.BlockSpec((RB, D), lambda i: (i, 0)),
            pl.BlockSpec((NCAND, RB, D), lambda i: (0, i, 0)),
            pl.BlockSpec((NCAND, RB), lambda i: (0, i)),
            pl.BlockSpec((1, RB), lambda i: (0, i)),
            pl.BlockSpec((1, RB), lambda i: (0, i)),
        ],
        out_specs=[
            pl.BlockSpec((RB, D), lambda i: (i, 0)),
            pl.BlockSpec((RB, D), lambda i: (i, 0)),
        ],
        out_shape=[
            jax.ShapeDtypeStruct((B, D), jnp.float32),
            jax.ShapeDtypeStruct((B, D), jnp.float32),
        ],
        interpret=interpret,
    )(query, cand, cols, rmaxt, rsumt)


def _member_cols(sel):
    g = jnp.clip(sel, 0, NSEGS - 1).astype(jnp.int32)
    base = (g // LANES) * CHUNK + (g % LANES)
    t = jnp.arange(SEG, dtype=jnp.int32)
    rows = base[:, None, :] + t[None, :, None] * LANES
    rows_t = rows.reshape(NCAND, B)
    valid = rows_t < M
    rows_flat = jnp.where(valid, rows_t, M - 1).reshape(NCAND * B)
    colsT = jnp.where(valid, rows_t.astype(jnp.float32), COL_SENTINEL)
    return rows_flat, colsT


def _half(query, m_items):
    sel, stat = _stats_topseg(query, m_items)
    rows_flat, colsT = _member_cols(sel)
    gathered = _sc_gather(m_items, rows_flat)
    cand = gathered.reshape(NCAND, B, D)
    return _combine(query, cand, colsT, stat[0:1, :], stat[1:2, :])


def kernel(query, m_items):
    o1a, o2a = _half(query[:B], m_items)
    o1b, o2b = _half(query[B:], m_items)
    return (jnp.concatenate([o1a, o1b], axis=0),
            jnp.concatenate([o2a, o2b], axis=0))
